# baseline (speedup 1.0000x reference)
import functools

import jax
import jax.numpy as jnp
from jax import lax
from jax.experimental import pallas as pl
from jax.experimental.pallas import tpu as pltpu
from jax.experimental.pallas import tpu_sc as plsc

N_NODES = 10000
E = 320000
COORD_DIM = 128
D_EDGE = 16
H = 512
OUT_DIM = 3


@functools.cache
def _make_sc_gather():
    info = plsc.get_sparse_core_info()
    nw = info.num_cores * info.num_subcores
    per_w = E // nw
    ch = 400
    n_ch = per_w // ch
    mesh = plsc.VectorSubcoreMesh(core_axis_name="c", subcore_axis_name="s")

    @functools.partial(
        pl.kernel,
        mesh=mesh,
        out_type=[
            jax.ShapeDtypeStruct((E, COORD_DIM), jnp.float32),
            jax.ShapeDtypeStruct((E, COORD_DIM), jnp.float32),
        ],
        scratch_types=[
            pltpu.VMEM((ch,), jnp.int32),
            pltpu.VMEM((ch, COORD_DIM), jnp.float32),
            pltpu.SemaphoreType.DMA,
        ],
    )
    def sc_gather(coord_hbm, src_hbm, dst_hbm, xi_hbm, xj_hbm, idx_v, rows_v, sem):
        wid = lax.axis_index("s") * info.num_cores + lax.axis_index("c")
        base = wid * per_w

        def body(c, carry):
            off = base + c * ch
            pltpu.sync_copy(src_hbm.at[pl.ds(off, ch)], idx_v)
            pltpu.async_copy(coord_hbm.at[idx_v], rows_v, sem).wait()
            pltpu.sync_copy(rows_v, xi_hbm.at[pl.ds(off, ch)])
            pltpu.sync_copy(dst_hbm.at[pl.ds(off, ch)], idx_v)
            pltpu.async_copy(coord_hbm.at[idx_v], rows_v, sem).wait()
            pltpu.sync_copy(rows_v, xj_hbm.at[pl.ds(off, ch)])
            return carry

        lax.fori_loop(0, n_ch, body, 0)

    return sc_gather


def _mlp_body(xi, xj, f, nf, w0a, w0b, w0c, b0, w1, b1, w2, b2, w3, b3, out):
    dot = functools.partial(jnp.dot, preferred_element_type=jnp.float32)
    xib = xi[...].astype(jnp.bfloat16)
    xjb = xj[...].astype(jnp.bfloat16)
    h = dot(xib, w0a[...]) + dot(xjb, w0b[...]) + dot(f[...], w0c[...])
    h = jnp.maximum(h + b0[...], 0.0).astype(jnp.bfloat16)
    h = jnp.maximum(dot(h, w1[...]) + b1[...], 0.0).astype(jnp.bfloat16)
    h = jnp.maximum(dot(h, w2[...]) + b2[...], 0.0).astype(jnp.bfloat16)
    out[...] = (dot(h, w3[...]) + b3[...]) * nf[...]


def _mlp_call(xi, xj, feat, nf, w0a, w0b, w0c, b0, w1, b1, w2, b2, w3, b3):
    blk = 512
    grid = (E // blk,)

    def row_spec(d):
        return pl.BlockSpec((blk, d), lambda i: (i, 0))

    def full_spec(shape):
        return pl.BlockSpec(shape, lambda i: (0,) * len(shape))

    return pl.pallas_call(
        _mlp_body,
        grid=grid,
        in_specs=[
            row_spec(COORD_DIM),
            row_spec(COORD_DIM),
            row_spec(D_EDGE),
            row_spec(1),
            full_spec(w0a.shape),
            full_spec(w0b.shape),
            full_spec(w0c.shape),
            full_spec(b0.shape),
            full_spec(w1.shape),
            full_spec(b1.shape),
            full_spec(w2.shape),
            full_spec(b2.shape),
            full_spec(w3.shape),
            full_spec(b3.shape),
        ],
        out_specs=pl.BlockSpec((blk, OUT_DIM), lambda i: (i, 0)),
        out_shape=jax.ShapeDtypeStruct((E, OUT_DIM), jnp.float32),
    )(xi, xj, feat, nf, w0a, w0b, w0c, b0, w1, b1, w2, b2, w3, b3)


def kernel(coordinates, feature_array, non_fictitious, src, dst,
           W0, b0, W1, b1, W2, b2, W3, b3):
    bf = jnp.bfloat16
    xi, xj = _make_sc_gather()(coordinates, src, dst)
    w0a = W0[:COORD_DIM].astype(bf)
    w0b = W0[COORD_DIM:2 * COORD_DIM].astype(bf)
    w0c = W0[2 * COORD_DIM:].astype(bf)
    nf = non_fictitious.reshape(E, 1)
    return _mlp_call(
        xi, xj, feature_array.astype(bf), nf,
        w0a, w0b, w0c, b0.reshape(1, H),
        W1.astype(bf), b1.reshape(1, H),
        W2.astype(bf), b2.reshape(1, H),
        W3.astype(bf), b3.reshape(1, OUT_DIM),
    )


# speedup vs baseline: 1.2897x; 1.2897x over previous
import functools

import jax
import jax.numpy as jnp
from jax import lax
from jax.experimental import pallas as pl
from jax.experimental.pallas import tpu as pltpu
from jax.experimental.pallas import tpu_sc as plsc

N_NODES = 10000
E = 320000
COORD_DIM = 128
D_EDGE = 16
H = 512
OUT_DIM = 3


@functools.cache
def _make_sc_gather():
    info = plsc.get_sparse_core_info()
    nw = info.num_cores * info.num_subcores
    per_w = E // nw
    ch = 400
    n_ch = per_w // ch
    mesh = plsc.VectorSubcoreMesh(core_axis_name="c", subcore_axis_name="s")

    @functools.partial(
        pl.kernel,
        mesh=mesh,
        out_type=jax.ShapeDtypeStruct((E, 2 * COORD_DIM), jnp.float32),
        scratch_types=[
            pltpu.VMEM((ch,), jnp.int32),
            pltpu.VMEM((ch, COORD_DIM), jnp.float32),
            pltpu.VMEM((ch,), jnp.int32),
            pltpu.VMEM((ch, COORD_DIM), jnp.float32),
            pltpu.SemaphoreType.DMA,
            pltpu.SemaphoreType.DMA,
        ],
    )
    def sc_gather(coord_hbm, src_hbm, dst_hbm, x_hbm,
                  idxa_v, rowsa_v, idxb_v, rowsb_v, sema, semb):
        wid = lax.axis_index("s") * info.num_cores + lax.axis_index("c")
        base = wid * per_w

        def body(c, carry):
            off = base + c * ch
            pltpu.sync_copy(src_hbm.at[pl.ds(off, ch)], idxa_v)
            pltpu.sync_copy(dst_hbm.at[pl.ds(off, ch)], idxb_v)
            ga = pltpu.async_copy(coord_hbm.at[idxa_v], rowsa_v, sema)
            gb = pltpu.async_copy(coord_hbm.at[idxb_v], rowsb_v, semb)
            ga.wait()
            wa = pltpu.async_copy(
                rowsa_v, x_hbm.at[pl.ds(off, ch), pl.ds(0, COORD_DIM)], sema)
            gb.wait()
            wb = pltpu.async_copy(
                rowsb_v, x_hbm.at[pl.ds(off, ch), pl.ds(COORD_DIM, COORD_DIM)],
                semb)
            wa.wait()
            wb.wait()
            return carry

        lax.fori_loop(0, n_ch, body, 0)

    return sc_gather


def _mlp_body(x, f, nf, w0ab, w0c, b0, w1, b1, w2, b2, w3, b3, out):
    bf = jnp.bfloat16
    dot = functools.partial(jnp.dot, preferred_element_type=jnp.float32)
    h = dot(x[...].astype(bf), w0ab[...]) + dot(f[...], w0c[...])
    h = jnp.maximum(h + b0[...], 0.0).astype(bf)
    h = jnp.maximum(dot(h, w1[...]) + b1[...], 0.0).astype(bf)
    h = jnp.maximum(dot(h, w2[...]) + b2[...], 0.0).astype(bf)
    out[...] = (dot(h, w3[...]) + b3[...]) * nf[...]


def _mlp_call(x, feat, nf, w0ab, w0c, b0, w1, b1, w2, b2, w3, b3):
    blk = 2560
    grid = (E // blk,)

    def row_spec(d):
        return pl.BlockSpec((blk, d), lambda i: (i, 0))

    def full_spec(shape):
        return pl.BlockSpec(shape, lambda i: (0,) * len(shape))

    return pl.pallas_call(
        _mlp_body,
        grid=grid,
        in_specs=[
            row_spec(2 * COORD_DIM),
            row_spec(D_EDGE),
            row_spec(1),
            full_spec(w0ab.shape),
            full_spec(w0c.shape),
            full_spec(b0.shape),
            full_spec(w1.shape),
            full_spec(b1.shape),
            full_spec(w2.shape),
            full_spec(b2.shape),
            full_spec(w3.shape),
            full_spec(b3.shape),
        ],
        out_specs=pl.BlockSpec((blk, OUT_DIM), lambda i: (i, 0)),
        out_shape=jax.ShapeDtypeStruct((E, OUT_DIM), jnp.float32),
    )(x, feat, nf, w0ab, w0c, b0, w1, b1, w2, b2, w3, b3)


def kernel(coordinates, feature_array, non_fictitious, src, dst,
           W0, b0, W1, b1, W2, b2, W3, b3):
    bf = jnp.bfloat16
    x = _make_sc_gather()(coordinates, src, dst)
    w0ab = W0[:2 * COORD_DIM].astype(bf)
    w0c = W0[2 * COORD_DIM:].astype(bf)
    nf = non_fictitious.reshape(E, 1)
    return _mlp_call(
        x, feature_array.astype(bf), nf,
        w0ab, w0c, b0.reshape(1, H),
        W1.astype(bf), b1.reshape(1, H),
        W2.astype(bf), b2.reshape(1, H),
        W3.astype(bf), b3.reshape(1, OUT_DIM),
    )
